# baseline (speedup 1.0000x reference)
import functools

import jax
import jax.numpy as jnp
from jax import lax
from jax.experimental import pallas as pl
from jax.experimental.pallas import tpu as pltpu
from jax.experimental.pallas import tpu_sc as plsc

_D = 64
_SCALE = 8.0
_NC, _NS = 2, 16
_NW = _NC * _NS
_B = 4096 * 200
_BPW = _B // _NW
_CHUNK = 128
_STEPS = _BPW // _CHUNK

_mesh = plsc.VectorSubcoreMesh(
    core_axis_name="c", subcore_axis_name="s",
    num_cores=_NC, num_subcores=_NS)


@functools.partial(
    pl.kernel,
    out_type=jax.ShapeDtypeStruct((_B, _D), jnp.float32),
    mesh=_mesh,
    scratch_types=[
        pltpu.VMEM((_CHUNK,), jnp.int32),
        pltpu.VMEM((_CHUNK, _D), jnp.float32),
        pltpu.SemaphoreType.DMA,
    ],
    compiler_params=pltpu.CompilerParams(use_tc_tiling_on_sc=False),
)
def _embed(word_hbm, table_hbm, out_hbm, idx_v, rows_v, sem):
    wid = lax.axis_index("s") * _NC + lax.axis_index("c")
    base = wid * _BPW

    def step(g, carry):
        off = base + g * _CHUNK
        pltpu.sync_copy(word_hbm.at[pl.ds(off, _CHUNK)], idx_v)
        pltpu.async_copy(table_hbm.at[idx_v], rows_v, sem).wait()

        def scale_row(i, c):
            for j in range(_D // 16):
                sl = pl.ds(j * 16, 16)
                rows_v[i, sl] = rows_v[i, sl] * _SCALE
            return c

        lax.fori_loop(0, _CHUNK, scale_row, 0)
        pltpu.sync_copy(rows_v, out_hbm.at[pl.ds(off, _CHUNK)])
        return carry

    lax.fori_loop(0, _STEPS, step, 0)


def kernel(word, table):
    flat = word.reshape(-1).astype(jnp.int32)
    out = _embed(flat, table)
    return out.reshape(word.shape[0], word.shape[1], _D)


# speedup vs baseline: 1.2663x; 1.2663x over previous
import functools

import jax
import jax.numpy as jnp
from jax import lax
from jax.experimental import pallas as pl
from jax.experimental.pallas import tpu as pltpu
from jax.experimental.pallas import tpu_sc as plsc

_D = 64
_SCALE = 8.0
_NC, _NS = 2, 16
_NW = _NC * _NS
_B = 4096 * 200
_CH = 128
_NCHUNK = _B // _CH
_CPT = _NCHUNK // _NW
_NBUF = 4

_mesh = plsc.VectorSubcoreMesh(
    core_axis_name="c", subcore_axis_name="s",
    num_cores=_NC, num_subcores=_NS)


@functools.partial(
    pl.kernel,
    out_type=jax.ShapeDtypeStruct((_NCHUNK, _CH, _D), jnp.float32),
    mesh=_mesh,
    scratch_types=[
        pltpu.VMEM((_NBUF, _CH), jnp.int32),
        pltpu.VMEM((_NBUF, _CH, _D), jnp.float32),
        pltpu.VMEM((_NBUF, _CH, _D), jnp.float32),
        pltpu.SemaphoreType.DMA((_NBUF,)),
        pltpu.SemaphoreType.DMA((_NBUF,)),
        pltpu.SemaphoreType.DMA((_NBUF,)),
    ],
    compiler_params=pltpu.CompilerParams(use_tc_tiling_on_sc=False),
)
def _embed(word_hbm, table_hbm, out_hbm, idx, rows, wb, sem_i, sem_g, sem_o):
    wid = lax.axis_index("s") * _NC + lax.axis_index("c")
    cbase = wid * _CPT

    for b in range(_NBUF):
        pltpu.sync_copy(word_hbm.at[cbase + b], idx.at[b])
        pltpu.async_copy(table_hbm.at[idx.at[b]], rows.at[b], sem_g.at[b])

    def outer(r, carry):
        for b in range(_NBUF):
            g = r * _NBUF + b
            more = r <= (_CPT // _NBUF) - 2

            pltpu.make_async_copy(
                table_hbm.at[idx.at[b]], rows.at[b], sem_g.at[b]).wait()

            @pl.when(more)
            def _():
                pltpu.async_copy(
                    word_hbm.at[cbase + g + _NBUF], idx.at[b], sem_i.at[b])

            @pl.when(r >= 1)
            def _():
                pltpu.make_async_copy(
                    wb.at[b], out_hbm.at[cbase + g - _NBUF], sem_o.at[b]).wait()

            def scale_rows(i, c):
                for j in range(_D // 16):
                    sl = pl.ds(j * 16, 16)
                    wb[b, i, sl] = rows[b, i, sl] * _SCALE
                return c

            lax.fori_loop(0, _CH, scale_rows, 0)

            @pl.when(more)
            def _():
                pltpu.make_async_copy(
                    word_hbm.at[cbase + g + _NBUF], idx.at[b], sem_i.at[b]).wait()
                pltpu.async_copy(
                    table_hbm.at[idx.at[b]], rows.at[b], sem_g.at[b])

            pltpu.async_copy(wb.at[b], out_hbm.at[cbase + g], sem_o.at[b])
        return carry

    lax.fori_loop(0, _CPT // _NBUF, outer, 0)

    for b in range(_NBUF):
        pltpu.make_async_copy(
            wb.at[b], out_hbm.at[cbase + _CPT - _NBUF + b], sem_o.at[b]).wait()


def kernel(word, table):
    flat = word.reshape(_NCHUNK, _CH).astype(jnp.int32)
    out = _embed(flat, table)
    return out.reshape(word.shape[0], word.shape[1], _D)


# speedup vs baseline: 1.2665x; 1.0001x over previous
import functools

import jax
import jax.numpy as jnp
from jax import lax
from jax.experimental import pallas as pl
from jax.experimental.pallas import tpu as pltpu
from jax.experimental.pallas import tpu_sc as plsc

_D = 64
_SCALE = 8.0
_NC, _NS = 2, 16
_NW = _NC * _NS
_B = 4096 * 200
_CH = 128
_NCHUNK = _B // _CH
_CPT = _NCHUNK // _NW
_NBUF = 4

_mesh = plsc.VectorSubcoreMesh(
    core_axis_name="c", subcore_axis_name="s",
    num_cores=_NC, num_subcores=_NS)


@functools.partial(
    pl.kernel,
    out_type=jax.ShapeDtypeStruct((_NCHUNK, _CH, _D), jnp.float32),
    mesh=_mesh,
    scratch_types=[
        pltpu.VMEM((_NBUF, _CH), jnp.int32),
        pltpu.VMEM((_NBUF, _CH, _D), jnp.float32),
        pltpu.VMEM((_NBUF, _CH, _D), jnp.float32),
        pltpu.SemaphoreType.DMA((_NBUF,)),
        pltpu.SemaphoreType.DMA((_NBUF,)),
        pltpu.SemaphoreType.DMA((_NBUF,)),
    ],
    compiler_params=pltpu.CompilerParams(use_tc_tiling_on_sc=False),
)
def _embed(word_hbm, table_hbm, out_hbm, idx, rows, wb, sem_i, sem_g, sem_o):
    wid = lax.axis_index("s") * _NC + lax.axis_index("c")
    cbase = wid * _CPT

    for b in range(_NBUF):
        pltpu.sync_copy(word_hbm.at[cbase + b], idx.at[b])
        pltpu.async_copy(table_hbm.at[idx.at[b]], rows.at[b], sem_g.at[b])

    def outer(r, carry):
        for b in range(_NBUF):
            g = r * _NBUF + b
            more = r <= (_CPT // _NBUF) - 2

            pltpu.make_async_copy(
                table_hbm.at[idx.at[b]], rows.at[b], sem_g.at[b]).wait()

            @pl.when(more)
            def _():
                pltpu.async_copy(
                    word_hbm.at[cbase + g + _NBUF], idx.at[b], sem_i.at[b])

            @pl.when(r >= 1)
            def _():
                pltpu.make_async_copy(
                    wb.at[b], out_hbm.at[cbase + g - _NBUF], sem_o.at[b]).wait()

            @functools.partial(plsc.parallel_loop, 0, _CH, unroll=4)
            def _(i):
                for j in range(_D // 16):
                    sl = pl.ds(j * 16, 16)
                    wb[b, i, sl] = rows[b, i, sl] * _SCALE

            @pl.when(more)
            def _():
                pltpu.make_async_copy(
                    word_hbm.at[cbase + g + _NBUF], idx.at[b], sem_i.at[b]).wait()
                pltpu.async_copy(
                    table_hbm.at[idx.at[b]], rows.at[b], sem_g.at[b])

            pltpu.async_copy(wb.at[b], out_hbm.at[cbase + g], sem_o.at[b])
        return carry

    lax.fori_loop(0, _CPT // _NBUF, outer, 0)

    for b in range(_NBUF):
        pltpu.make_async_copy(
            wb.at[b], out_hbm.at[cbase + _CPT - _NBUF + b], sem_o.at[b]).wait()


def kernel(word, table):
    flat = word.reshape(_NCHUNK, _CH).astype(jnp.int32)
    out = _embed(flat, table)
    return out.reshape(word.shape[0], word.shape[1], _D)


# speedup vs baseline: 1.6191x; 1.2784x over previous
import functools

import jax
import jax.numpy as jnp
from jax import lax
from jax.experimental import pallas as pl
from jax.experimental.pallas import tpu as pltpu
from jax.experimental.pallas import tpu_sc as plsc

_V = 1000000
_D = 64
_SCALE = 8.0
_NC, _NS = 2, 16
_NW = _NC * _NS
_NB = 4096
_NT = 200
_CH = 128
_NBB = _NB // _CH
_NCHUNK = _NT * _NBB
_CPT = _NCHUNK // _NW
_NBUF = 4

_mesh = plsc.VectorSubcoreMesh(
    core_axis_name="c", subcore_axis_name="s",
    num_cores=_NC, num_subcores=_NS)


@functools.partial(
    pl.kernel,
    out_type=jax.ShapeDtypeStruct((_NT, _D, _NB), jnp.float32),
    mesh=_mesh,
    scratch_types=[
        pltpu.VMEM((_NBUF, _CH), jnp.int32),
        pltpu.VMEM((_CH, _D), jnp.float32),
        pltpu.VMEM((_CH, _D), jnp.float32),
        pltpu.VMEM((_CH, _D), jnp.float32),
        pltpu.VMEM((_CH, _D), jnp.float32),
        pltpu.VMEM((_NBUF, _D, _CH), jnp.float32),
        pltpu.SemaphoreType.DMA((_NBUF,)),
        pltpu.SemaphoreType.DMA((_NBUF,)),
        pltpu.SemaphoreType.DMA((_NBUF,)),
    ],
    compiler_params=pltpu.CompilerParams(use_tc_tiling_on_sc=False),
)
def _embed(word_hbm, table_hbm, out_hbm, idx, rows0, rows1, rows2, rows3,
           wb, sem_i, sem_g, sem_o):
    rows = [rows0, rows1, rows2, rows3]
    wid = lax.axis_index("s") * _NC + lax.axis_index("c")
    qbase = wid * _CPT
    iot = lax.iota(jnp.int32, 16)

    def widx(q):
        return pl.ds((q // _NBB) * _NB + (q % _NBB) * _CH, _CH)

    def wdst(q, buf):
        return out_hbm.at[q // _NBB, :, pl.ds((q % _NBB) * _CH, _CH)]

    for b in range(_NBUF):
        pltpu.sync_copy(word_hbm.at[widx(qbase + b)], idx.at[b])
        pltpu.async_copy(table_hbm.at[idx.at[b]], rows[b], sem_g.at[b])

    def outer(r, carry):
        for b in range(_NBUF):
            q = qbase + r * _NBUF + b
            more = r <= (_CPT // _NBUF) - 2

            pltpu.make_async_copy(
                table_hbm.at[idx.at[b]], rows[b], sem_g.at[b]).wait()

            @pl.when(more)
            def _():
                pltpu.async_copy(
                    word_hbm.at[widx(q + _NBUF)], idx.at[b], sem_i.at[b])

            @pl.when(r >= 1)
            def _():
                pltpu.make_async_copy(
                    wb.at[b], wdst(q - _NBUF, b), sem_o.at[b]).wait()

            @functools.partial(plsc.parallel_loop, 0, _D, unroll=2)
            def _(d):
                col = jnp.full((16,), d, jnp.int32)
                for bg in range(_CH // 16):
                    v = plsc.load_gather(
                        rows[b], [iot + bg * 16, col])
                    wb[b, d, pl.ds(bg * 16, 16)] = v * _SCALE

            @pl.when(more)
            def _():
                pltpu.make_async_copy(
                    word_hbm.at[widx(q + _NBUF)], idx.at[b], sem_i.at[b]).wait()
                pltpu.async_copy(
                    table_hbm.at[idx.at[b]], rows[b], sem_g.at[b])

            pltpu.async_copy(wb.at[b], wdst(q, b), sem_o.at[b])
        return carry

    lax.fori_loop(0, _CPT // _NBUF, outer, 0)

    for b in range(_NBUF):
        pltpu.make_async_copy(
            wb.at[b], wdst(qbase + _CPT - _NBUF + b, b), sem_o.at[b]).wait()


def kernel(word, table):
    word_t = word.T.reshape(-1).astype(jnp.int32)
    out_t = _embed(word_t, table)
    return out_t.transpose(2, 0, 1)
